# baseline (speedup 1.0000x reference)
import jax
import jax.numpy as jnp
from jax import lax
from jax.experimental import pallas as pl
from jax.experimental.pallas import tpu as pltpu, tpu_sc as plsc

L1 = 1024
NPSQT = 8
NLS = 8
NRF = 16
TAU = 1.0
MAX_FT_ACT = 1.0
L0_CORR = 127.0 / 128.0

D = L1 + NPSQT
D_PAD = 1152
K = 32
NW = 32
NBUF = 3


def _ft_bag_kernel(table_hbm, idx_hbm, val_hbm, out_hbm,
                   idx_v, val_v, rows0, rows1, rows2,
                   ostage0, ostage1, ostage2,
                   si0, si1, si2, so0, so1, so2):
    nb = idx_hbm.shape[0] // K
    epw = nb // NW
    wid = lax.axis_index("c") * 16 + lax.axis_index("s")
    base = wid * epw

    pltpu.sync_copy(idx_hbm.at[pl.ds(base * K, epw * K)], idx_v)
    pltpu.sync_copy(val_hbm.at[pl.ds(base * K, epw * K)], val_v)

    rows = (rows0, rows1, rows2)
    ostage = (ostage0, ostage1, ostage2)
    sems_in = (si0, si1, si2)
    sems_out = (so0, so1, so2)

    def gather_copy(e, slot):
        isl = idx_v.at[pl.ds(pl.multiple_of(e * K, 16), K)]
        return pltpu.make_async_copy(
            table_hbm.at[isl], rows[slot], sems_in[slot])

    def out_copy(e, slot):
        return pltpu.make_async_copy(
            ostage[slot].at[pl.ds(0, D)],
            out_hbm.at[pl.ds((base + e) * D, D)], sems_out[slot])

    for s in range(NBUF):
        gather_copy(s, s).start()

    def do_example(e, slot):
        gather_copy(e, slot).wait()
        vv0 = val_v[pl.ds(pl.multiple_of(e * K, 16), 16)]
        vv1 = val_v[pl.ds(pl.multiple_of(e * K + 16, 16), 16)]
        vb = [jnp.full((16,), vv0[k] if k < 16 else vv1[k - 16], jnp.float32)
              for k in range(K)]

        def accum(off):
            acc = rows[slot][0, pl.ds(off, 16)] * vb[0]
            for k in range(1, K):
                acc = acc + rows[slot][k, pl.ds(off, 16)] * vb[k]
            ostage[slot][pl.ds(off, 16)] = acc

        def chunk_body(c, _):
            for j in range(4):
                accum(pl.multiple_of(c * 64 + j * 16, 16))
            return 0

        @pl.when(e >= NBUF)
        def _():
            out_copy(e - NBUF, slot).wait()

        lax.fori_loop(0, 16, chunk_body, 0)
        accum(1024)

        out_copy(e, slot).start()

        @pl.when(e + NBUF < epw)
        def _():
            gather_copy(e + NBUF, slot).start()

    def outer(g, _):
        e0 = g * NBUF
        for s in range(NBUF):
            do_example(e0 + s, s)
        return 0

    lax.fori_loop(0, (epw - 1) // NBUF, outer, 0)
    do_example(epw - 1, 0)

    out_copy(epw - 3, 1).wait()
    out_copy(epw - 2, 2).wait()
    out_copy(epw - 1, 0).wait()


def _ft_bag(table_pad, idx_flat, val_flat):
    nb = idx_flat.shape[0] // K
    epw = nb // NW
    mesh = plsc.VectorSubcoreMesh(core_axis_name="c", subcore_axis_name="s")
    return pl.kernel(
        _ft_bag_kernel,
        out_type=jax.ShapeDtypeStruct((nb * D,), jnp.float32),
        mesh=mesh,
        scratch_types=[
            pltpu.VMEM((epw * K,), jnp.int32),
            pltpu.VMEM((epw * K,), jnp.float32),
            pltpu.VMEM((K, D_PAD), jnp.float32),
            pltpu.VMEM((K, D_PAD), jnp.float32),
            pltpu.VMEM((K, D_PAD), jnp.float32),
            pltpu.VMEM((1040,), jnp.float32),
            pltpu.VMEM((1040,), jnp.float32),
            pltpu.VMEM((1040,), jnp.float32),
            pltpu.SemaphoreType.DMA,
            pltpu.SemaphoreType.DMA,
            pltpu.SemaphoreType.DMA,
            pltpu.SemaphoreType.DMA,
            pltpu.SemaphoreType.DMA,
            pltpu.SemaphoreType.DMA,
        ],
    )(table_pad, idx_flat, val_flat)


def _dense_kernel(accw, accb, us, them, g, ftb, rW, rb, rls,
                  W1T, b1f, W2bd, b2f, W3bd, b3f, out):
    wp = accw[...] + ftb[...]
    bp = accb[...] + ftb[...]
    w = wp[:, :L1]
    wps = wp[:, L1:]
    b_ = bp[:, :L1]
    bps = bp[:, L1:]
    u = us[...]
    t = them[...]
    l0w = jnp.clip(u * w + t * b_, 0.0, MAX_FT_ACT)
    l0b = jnp.clip(u * b_ + t * w, 0.0, MAX_FT_ACT)
    half = L1 // 2
    p0 = l0w[:, :half] * l0w[:, half:]
    p1 = l0b[:, :half] * l0b[:, half:]
    l0_ = jnp.concatenate([p0, p1], axis=1) * L0_CORR
    rf = jnp.concatenate([p0[:, half - NRF:], p1[:, half - NRF:]], axis=1)
    logits = rls[0, 0] * (
        jnp.dot(rf, rW[...], preferred_element_type=jnp.float32) + rb[...]
    )
    z = logits + g[...]
    zmax = jnp.max(z, axis=1, keepdims=True)
    iota8 = lax.broadcasted_iota(jnp.int32, z.shape, 1)
    first = jnp.min(jnp.where(z >= zmax, iota8, NLS), axis=1, keepdims=True)
    rw = (iota8 == first).astype(jnp.float32)
    h1 = jnp.clip(
        jnp.dot(l0_, W1T[...], preferred_element_type=jnp.float32) + b1f[...],
        0.0, 1.0)
    h2 = jnp.clip(
        jnp.dot(h1, W2bd[...], preferred_element_type=jnp.float32) + b2f[...],
        0.0, 1.0)
    oe = jnp.dot(h2, W3bd[...], preferred_element_type=jnp.float32) + b3f[...]
    x = jnp.sum(oe * rw, axis=1, keepdims=True)
    psqt = jnp.sum((wps - bps) * rw, axis=1, keepdims=True)
    out[...] = x + psqt * (u - 0.5)


def kernel(us, them, white_indices, white_values, black_indices, black_values,
           psqt_indices, layer_stack_indices, ft_W, ft_b, router_W, router_b,
           router_ls, W1, b1, W2, b2, W3, b3):
    B = us.shape[0]
    idx_flat = jnp.concatenate(
        [white_indices, black_indices], axis=0).astype(jnp.int32).reshape(-1)
    val_flat = jnp.concatenate(
        [white_values, black_values], axis=0).reshape(-1)

    table_pad = jnp.pad(ft_W, ((0, 0), (0, D_PAD - D)))
    acc = _ft_bag(table_pad, idx_flat, val_flat).reshape(2 * B, D)

    u = jax.random.uniform(jax.random.key(42), (B, NLS),
                           minval=1e-6, maxval=1.0 - 1e-6)
    gnoise = -jnp.log(-jnp.log(u)) / TAU

    L2d = W2.shape[1]
    W1T = W1.reshape(NLS * W1.shape[1], L1).T
    b1f = b1.reshape(1, -1)
    e_ids = jnp.arange(NLS)
    W2bd = jnp.zeros((NLS * W2.shape[2], NLS * L2d), jnp.float32)
    W2bd = W2bd.at[
        (e_ids[:, None, None] * W2.shape[2]
         + jnp.arange(W2.shape[2])[None, :, None]),
        (e_ids[:, None, None] * L2d + jnp.arange(L2d)[None, None, :]),
    ].set(jnp.transpose(W2, (0, 2, 1)))
    b2f = b2.reshape(1, -1)
    W3bd = jnp.zeros((NLS * L2d, NLS), jnp.float32)
    W3bd = W3bd.at[
        (e_ids[:, None] * L2d + jnp.arange(L2d)[None, :]),
        e_ids[:, None],
    ].set(W3[:, 0, :])
    b3f = b3.reshape(1, -1)

    BLK = 512
    nblk = B // BLK
    grid = (nblk,)
    z2 = lambda i: (i, 0)
    full = lambda i: (0, 0)
    out = pl.pallas_call(
        _dense_kernel,
        grid=grid,
        in_specs=[
            pl.BlockSpec((BLK, D), z2),
            pl.BlockSpec((BLK, D), lambda i: (i + nblk, 0)),
            pl.BlockSpec((BLK, 1), z2),
            pl.BlockSpec((BLK, 1), z2),
            pl.BlockSpec((BLK, NLS), z2),
            pl.BlockSpec((1, D), full),
            pl.BlockSpec((2 * NRF, NLS), full),
            pl.BlockSpec((1, NLS), full),
            pl.BlockSpec((1, 1), full),
            pl.BlockSpec((L1, NLS * 16), full),
            pl.BlockSpec((1, NLS * 16), full),
            pl.BlockSpec((NLS * 16, NLS * 32), full),
            pl.BlockSpec((1, NLS * 32), full),
            pl.BlockSpec((NLS * 32, NLS), full),
            pl.BlockSpec((1, NLS), full),
        ],
        out_specs=pl.BlockSpec((BLK, 1), z2),
        out_shape=jax.ShapeDtypeStruct((B, 1), jnp.float32),
    )(acc, acc, us, them, gnoise, ft_b.reshape(1, D), router_W,
      router_b.reshape(1, NLS), router_ls.reshape(1, 1), W1T, b1f,
      W2bd, b2f, W3bd, b3f)
    return out


# speedup vs baseline: 1.4356x; 1.4356x over previous
import jax
import jax.numpy as jnp
from jax import lax
from jax.experimental import pallas as pl
from jax.experimental.pallas import tpu as pltpu, tpu_sc as plsc

L1 = 1024
NPSQT = 8
NLS = 8
NRF = 16
TAU = 1.0
MAX_FT_ACT = 1.0
L0_CORR = 127.0 / 128.0

D = L1 + NPSQT
K = 32
NW = 32
NBUF = 3


def _ft_bag_kernel(table_hbm, idx_hbm, val_hbm, out_hbm,
                   idx_v, val_v, rows0, rows1, rows2,
                   ostage0, ostage1, ostage2,
                   si0, si1, si2, so0, so1, so2):
    nb = idx_hbm.shape[0] // K
    epw = nb // NW
    wid = lax.axis_index("c") * 16 + lax.axis_index("s")
    base = wid * epw

    pltpu.sync_copy(idx_hbm.at[pl.ds(base * K, epw * K)], idx_v)
    pltpu.sync_copy(val_hbm.at[pl.ds(base * K, epw * K)], val_v)

    rows = (rows0, rows1, rows2)
    ostage = (ostage0, ostage1, ostage2)
    sems_in = (si0, si1, si2)
    sems_out = (so0, so1, so2)

    def gather_copy(e, slot):
        isl = idx_v.at[pl.ds(pl.multiple_of(e * K, 16), K)]
        return pltpu.make_async_copy(
            table_hbm.at[isl], rows[slot], sems_in[slot])

    def out_copy(e, slot):
        return pltpu.make_async_copy(
            ostage[slot].at[pl.ds(0, D)],
            out_hbm.at[pl.ds((base + e) * D, D)], sems_out[slot])

    for s in range(NBUF):
        gather_copy(s, s).start()

    def do_example(e, slot):
        gather_copy(e, slot).wait()
        vv0 = val_v[pl.ds(pl.multiple_of(e * K, 16), 16)]
        vv1 = val_v[pl.ds(pl.multiple_of(e * K + 16, 16), 16)]
        vb = [jnp.full((16,), vv0[k] if k < 16 else vv1[k - 16], jnp.float32)
              for k in range(K)]

        def accum(off):
            acc = rows[slot][0, pl.ds(off, 16)] * vb[0]
            for k in range(1, K):
                acc = acc + rows[slot][k, pl.ds(off, 16)] * vb[k]
            ostage[slot][pl.ds(off, 16)] = acc

        def chunk_body(c, _):
            for j in range(4):
                accum(pl.multiple_of(c * 64 + j * 16, 16))
            return 0

        @pl.when(e >= NBUF)
        def _():
            out_copy(e - NBUF, slot).wait()

        lax.fori_loop(0, 16, chunk_body, 0)
        accum(D - 16)

        out_copy(e, slot).start()

        @pl.when(e + NBUF < epw)
        def _():
            gather_copy(e + NBUF, slot).start()

    def outer(g, _):
        e0 = g * NBUF
        for s in range(NBUF):
            do_example(e0 + s, s)
        return 0

    lax.fori_loop(0, (epw - 1) // NBUF, outer, 0)
    do_example(epw - 1, 0)

    out_copy(epw - 3, 1).wait()
    out_copy(epw - 2, 2).wait()
    out_copy(epw - 1, 0).wait()


def _ft_bag(table, idx_flat, val_flat):
    nb = idx_flat.shape[0] // K
    epw = nb // NW
    mesh = plsc.VectorSubcoreMesh(core_axis_name="c", subcore_axis_name="s")
    return pl.kernel(
        _ft_bag_kernel,
        out_type=jax.ShapeDtypeStruct((nb * D,), jnp.float32),
        mesh=mesh,
        compiler_params=pltpu.CompilerParams(use_tc_tiling_on_sc=False),
        scratch_types=[
            pltpu.VMEM((epw * K,), jnp.int32),
            pltpu.VMEM((epw * K,), jnp.float32),
            pltpu.VMEM((K, D), jnp.float32),
            pltpu.VMEM((K, D), jnp.float32),
            pltpu.VMEM((K, D), jnp.float32),
            pltpu.VMEM((D,), jnp.float32),
            pltpu.VMEM((D,), jnp.float32),
            pltpu.VMEM((D,), jnp.float32),
            pltpu.SemaphoreType.DMA,
            pltpu.SemaphoreType.DMA,
            pltpu.SemaphoreType.DMA,
            pltpu.SemaphoreType.DMA,
            pltpu.SemaphoreType.DMA,
            pltpu.SemaphoreType.DMA,
        ],
    )(table, idx_flat, val_flat)


def _dense_kernel(accw, accb, us, them, g, ftb, rW, rb, rls,
                  W1T, b1f, W2bd, b2f, W3bd, b3f, out):
    wp = accw[...] + ftb[...]
    bp = accb[...] + ftb[...]
    w = wp[:, :L1]
    wps = wp[:, L1:]
    b_ = bp[:, :L1]
    bps = bp[:, L1:]
    u = us[...]
    t = them[...]
    l0w = jnp.clip(u * w + t * b_, 0.0, MAX_FT_ACT)
    l0b = jnp.clip(u * b_ + t * w, 0.0, MAX_FT_ACT)
    half = L1 // 2
    p0 = l0w[:, :half] * l0w[:, half:]
    p1 = l0b[:, :half] * l0b[:, half:]
    l0_ = jnp.concatenate([p0, p1], axis=1) * L0_CORR
    rf = jnp.concatenate([p0[:, half - NRF:], p1[:, half - NRF:]], axis=1)
    logits = rls[0, 0] * (
        jnp.dot(rf, rW[...], preferred_element_type=jnp.float32) + rb[...]
    )
    z = logits + g[...]
    zmax = jnp.max(z, axis=1, keepdims=True)
    iota8 = lax.broadcasted_iota(jnp.int32, z.shape, 1)
    first = jnp.min(jnp.where(z >= zmax, iota8, NLS), axis=1, keepdims=True)
    rw = (iota8 == first).astype(jnp.float32)
    h1 = jnp.clip(
        jnp.dot(l0_, W1T[...], preferred_element_type=jnp.float32) + b1f[...],
        0.0, 1.0)
    h2 = jnp.clip(
        jnp.dot(h1, W2bd[...], preferred_element_type=jnp.float32) + b2f[...],
        0.0, 1.0)
    oe = jnp.dot(h2, W3bd[...], preferred_element_type=jnp.float32) + b3f[...]
    x = jnp.sum(oe * rw, axis=1, keepdims=True)
    psqt = jnp.sum((wps - bps) * rw, axis=1, keepdims=True)
    out[...] = x + psqt * (u - 0.5)


def kernel(us, them, white_indices, white_values, black_indices, black_values,
           psqt_indices, layer_stack_indices, ft_W, ft_b, router_W, router_b,
           router_ls, W1, b1, W2, b2, W3, b3):
    B = us.shape[0]
    idx_flat = jnp.concatenate(
        [white_indices, black_indices], axis=0).astype(jnp.int32).reshape(-1)
    val_flat = jnp.concatenate(
        [white_values, black_values], axis=0).reshape(-1)

    acc = _ft_bag(ft_W, idx_flat, val_flat).reshape(2 * B, D)

    u = jax.random.uniform(jax.random.key(42), (B, NLS),
                           minval=1e-6, maxval=1.0 - 1e-6)
    gnoise = -jnp.log(-jnp.log(u)) / TAU

    L2d = W2.shape[1]
    W1T = W1.reshape(NLS * W1.shape[1], L1).T
    b1f = b1.reshape(1, -1)
    e_ids = jnp.arange(NLS)
    W2bd = jnp.zeros((NLS * W2.shape[2], NLS * L2d), jnp.float32)
    W2bd = W2bd.at[
        (e_ids[:, None, None] * W2.shape[2]
         + jnp.arange(W2.shape[2])[None, :, None]),
        (e_ids[:, None, None] * L2d + jnp.arange(L2d)[None, None, :]),
    ].set(jnp.transpose(W2, (0, 2, 1)))
    b2f = b2.reshape(1, -1)
    W3bd = jnp.zeros((NLS * L2d, NLS), jnp.float32)
    W3bd = W3bd.at[
        (e_ids[:, None] * L2d + jnp.arange(L2d)[None, :]),
        e_ids[:, None],
    ].set(W3[:, 0, :])
    b3f = b3.reshape(1, -1)

    BLK = 512
    nblk = B // BLK
    grid = (nblk,)
    z2 = lambda i: (i, 0)
    full = lambda i: (0, 0)
    out = pl.pallas_call(
        _dense_kernel,
        grid=grid,
        in_specs=[
            pl.BlockSpec((BLK, D), z2),
            pl.BlockSpec((BLK, D), lambda i: (i + nblk, 0)),
            pl.BlockSpec((BLK, 1), z2),
            pl.BlockSpec((BLK, 1), z2),
            pl.BlockSpec((BLK, NLS), z2),
            pl.BlockSpec((1, D), full),
            pl.BlockSpec((2 * NRF, NLS), full),
            pl.BlockSpec((1, NLS), full),
            pl.BlockSpec((1, 1), full),
            pl.BlockSpec((L1, NLS * 16), full),
            pl.BlockSpec((1, NLS * 16), full),
            pl.BlockSpec((NLS * 16, NLS * 32), full),
            pl.BlockSpec((1, NLS * 32), full),
            pl.BlockSpec((NLS * 32, NLS), full),
            pl.BlockSpec((1, NLS), full),
        ],
        out_specs=pl.BlockSpec((BLK, 1), z2),
        out_shape=jax.ShapeDtypeStruct((B, 1), jnp.float32),
    )(acc, acc, us, them, gnoise, ft_b.reshape(1, D), router_W,
      router_b.reshape(1, NLS), router_ls.reshape(1, 1), W1T, b1f,
      W2bd, b2f, W3bd, b3f)
    return out
